# Initial kernel scaffold; baseline (speedup 1.0000x reference)
import jax
import jax.numpy as jnp
from jax.experimental import pallas as pl


def kernel(x, W1, W2, W3, G1, G2, G3):
    raise NotImplementedError("write your pallas kernel here")


# speedup vs baseline: 1.3195x; 1.3195x over previous
import functools

import jax
import jax.numpy as jnp
from jax.experimental import pallas as pl

Q = 1024
D_IN = 512
D_EMB = 128
K_GAL = 20000

TAU = 1.75
ALPHA = 0.4
INV_TAU2 = 1.0 / (TAU * TAU)

KBLK = 2048


def _embed_kernel(x_ref, w1_ref, w2_ref, w3_ref, e1_ref, qn2_ref, qn3_ref):
    x = x_ref[...]
    e1_ref[...] = jax.lax.dot_general(
        x, w1_ref[...], (((1,), (0,)), ((), ())),
        preferred_element_type=jnp.float32)
    e2 = jax.lax.dot_general(
        x, w2_ref[...], (((1,), (0,)), ((), ())),
        preferred_element_type=jnp.float32)
    e3 = jax.lax.dot_general(
        x, w3_ref[...], (((1,), (0,)), ((), ())),
        preferred_element_type=jnp.float32)
    n2 = jnp.sqrt(jnp.sum(e2 * e2, axis=1, keepdims=True)) + 1e-12
    qn2_ref[...] = e2 / n2
    n3 = jnp.sqrt(jnp.sum(e3 * e3, axis=1, keepdims=True)) + 1e-12
    qn3_ref[...] = e3 / n3


def _main_kernel(e1_ref, qn2_ref, qn3_ref, g1_ref, g2_ref, g3_ref, o_ref):
    e1 = e1_ref[...]
    g1 = g1_ref[...]
    q2 = jnp.sum(e1 * e1, axis=1, keepdims=True)
    g1sq = jnp.sum(g1 * g1, axis=1)[None, :]
    s1 = jax.lax.dot_general(
        e1, g1, (((1,), (1,)), ((), ())), preferred_element_type=jnp.float32)
    d2 = jnp.maximum(q2 + g1sq - 2.0 * s1, 0.0)
    t = (d2 + 1e-12) * INV_TAU2
    sim = jnp.exp(-(t * t))
    cer = jnp.where(sim >= ALPHA, sim, 0.0)

    g2 = g2_ref[...]
    gn2 = g2 * (1.0 / (jnp.sqrt(jnp.sum(g2 * g2, axis=1, keepdims=True)) + 1e-12))
    c2 = jax.lax.dot_general(
        qn2_ref[...], gn2, (((1,), (1,)), ((), ())),
        preferred_element_type=jnp.float32)

    g3 = g3_ref[...]
    gn3 = g3 * (1.0 / (jnp.sqrt(jnp.sum(g3 * g3, axis=1, keepdims=True)) + 1e-12))
    c3 = jax.lax.dot_general(
        qn3_ref[...], gn3, (((1,), (1,)), ((), ())),
        preferred_element_type=jnp.float32)

    o_ref[...] = (cer + c2 + c3) * (1.0 / 3.0)


@functools.partial(jax.jit, static_argnames=("interpret",))
def kernel(x, W1, W2, W3, G1, G2, G3, interpret=False):
    e1, qn2, qn3 = pl.pallas_call(
        _embed_kernel,
        out_shape=[jax.ShapeDtypeStruct((Q, D_EMB), jnp.float32)] * 3,
        interpret=interpret,
    )(x, W1, W2, W3)

    nblk = pl.cdiv(K_GAL, KBLK)
    gal_spec = pl.BlockSpec((KBLK, D_EMB), lambda k: (k, 0))
    emb_spec = pl.BlockSpec((Q, D_EMB), lambda k: (0, 0))
    out = pl.pallas_call(
        _main_kernel,
        grid=(nblk,),
        in_specs=[emb_spec, emb_spec, emb_spec, gal_spec, gal_spec, gal_spec],
        out_specs=pl.BlockSpec((Q, KBLK), lambda k: (0, k)),
        out_shape=jax.ShapeDtypeStruct((Q, K_GAL), jnp.float32),
        interpret=interpret,
    )(e1, qn2, qn3, G1, G2, G3)
    return out


# speedup vs baseline: 1.5045x; 1.1402x over previous
import functools
import math

import jax
import jax.numpy as jnp
from jax.experimental import pallas as pl

Q = 1024
D_IN = 512
D_EMB = 128
K_GAL = 20000

TAU = 1.75
ALPHA = 0.4
S_SCALE = math.sqrt(math.log2(math.e)) / (TAU * TAU)
S_EPS = S_SCALE * 1e-12
C_THIRD = -math.log2(3.0)
T_CUT = math.log2(ALPHA) + C_THIRD

KBLK = 2048


def _embed_kernel(x_ref, w1_ref, w2_ref, w3_ref, e1s_ref, qb_ref, qc_ref):
    x = x_ref[...]
    e1 = jax.lax.dot_general(
        x, w1_ref[...], (((1,), (0,)), ((), ())),
        preferred_element_type=jnp.float32)
    q2 = jnp.sum(e1 * e1, axis=1, keepdims=True)
    e1s_ref[...] = e1 * (-2.0 * S_SCALE)
    qb_ref[...] = S_SCALE * q2 + S_EPS
    e2 = jax.lax.dot_general(
        x, w2_ref[...], (((1,), (0,)), ((), ())),
        preferred_element_type=jnp.float32)
    e3 = jax.lax.dot_general(
        x, w3_ref[...], (((1,), (0,)), ((), ())),
        preferred_element_type=jnp.float32)
    qn2 = e2 * ((1.0 / 3.0) / (jnp.sqrt(jnp.sum(e2 * e2, axis=1, keepdims=True)) + 1e-12))
    qn3 = e3 * ((1.0 / 3.0) / (jnp.sqrt(jnp.sum(e3 * e3, axis=1, keepdims=True)) + 1e-12))
    qc_ref[...] = jnp.concatenate([qn2, qn3], axis=1)


def _main_kernel(e1s_ref, qb_ref, qc_ref, g1_ref, g2_ref, g3_ref, o_ref):
    g1 = g1_ref[...]
    g1b = S_SCALE * jnp.sum(g1 * g1, axis=1)[None, :]
    m0 = jax.lax.dot_general(
        e1s_ref[...], g1, (((1,), (1,)), ((), ())),
        preferred_element_type=jnp.float32)
    w = jnp.maximum(m0 + qb_ref[...] + g1b, S_EPS)
    t = C_THIRD - w * w
    cer3 = jnp.where(t >= T_CUT, jnp.exp2(t), 0.0)

    g2 = g2_ref[...]
    g3 = g3_ref[...]
    r2 = 1.0 / (jnp.sqrt(jnp.sum(g2 * g2, axis=1, keepdims=True)) + 1e-12)
    r3 = 1.0 / (jnp.sqrt(jnp.sum(g3 * g3, axis=1, keepdims=True)) + 1e-12)
    gc = jnp.concatenate([g2 * r2, g3 * r3], axis=1)
    ccos = jax.lax.dot_general(
        qc_ref[...], gc, (((1,), (1,)), ((), ())),
        preferred_element_type=jnp.float32)
    o_ref[...] = cer3 + ccos


@functools.partial(jax.jit, static_argnames=("interpret",))
def kernel(x, W1, W2, W3, G1, G2, G3, interpret=False):
    e1s, qb, qc = pl.pallas_call(
        _embed_kernel,
        out_shape=[
            jax.ShapeDtypeStruct((Q, D_EMB), jnp.float32),
            jax.ShapeDtypeStruct((Q, 1), jnp.float32),
            jax.ShapeDtypeStruct((Q, 2 * D_EMB), jnp.float32),
        ],
        interpret=interpret,
    )(x, W1, W2, W3)

    nblk = pl.cdiv(K_GAL, KBLK)
    gal_spec = pl.BlockSpec((KBLK, D_EMB), lambda k: (k, 0))
    out = pl.pallas_call(
        _main_kernel,
        grid=(nblk,),
        in_specs=[
            pl.BlockSpec((Q, D_EMB), lambda k: (0, 0)),
            pl.BlockSpec((Q, 1), lambda k: (0, 0)),
            pl.BlockSpec((Q, 2 * D_EMB), lambda k: (0, 0)),
            gal_spec, gal_spec, gal_spec,
        ],
        out_specs=pl.BlockSpec((Q, KBLK), lambda k: (0, k)),
        out_shape=jax.ShapeDtypeStruct((Q, K_GAL), jnp.float32),
        interpret=interpret,
    )(e1s, qb, qc, G1, G2, G3)
    return out


# speedup vs baseline: 1.5421x; 1.0250x over previous
import functools
import math

import jax
import jax.numpy as jnp
from jax.experimental import pallas as pl
from jax.experimental.pallas import tpu as pltpu

Q = 1024
D_IN = 512
D_EMB = 128
K_GAL = 20000

TAU = 1.75
ALPHA = 0.4
S_SCALE = math.sqrt(math.log2(math.e)) / (TAU * TAU)
S_EPS = S_SCALE * 1e-12
C_THIRD = -math.log2(3.0)
T_CUT = math.log2(ALPHA) + C_THIRD

KBLK = 2048


def _embed_kernel(x_ref, w1_ref, w2_ref, w3_ref, e1s_ref, qb_ref, qc_ref):
    x = x_ref[...]
    e1 = jax.lax.dot_general(
        x, w1_ref[...], (((1,), (0,)), ((), ())),
        preferred_element_type=jnp.float32)
    q2 = jnp.sum(e1 * e1, axis=1, keepdims=True)
    e1s_ref[...] = e1 * (-2.0 * S_SCALE)
    qb_ref[...] = S_SCALE * q2 + S_EPS
    e2 = jax.lax.dot_general(
        x, w2_ref[...], (((1,), (0,)), ((), ())),
        preferred_element_type=jnp.float32)
    e3 = jax.lax.dot_general(
        x, w3_ref[...], (((1,), (0,)), ((), ())),
        preferred_element_type=jnp.float32)
    qn2 = e2 * ((1.0 / 3.0) / (jnp.sqrt(jnp.sum(e2 * e2, axis=1, keepdims=True)) + 1e-12))
    qn3 = e3 * ((1.0 / 3.0) / (jnp.sqrt(jnp.sum(e3 * e3, axis=1, keepdims=True)) + 1e-12))
    qc_ref[...] = jnp.concatenate([qn2, qn3], axis=1)


def _main_kernel(e1s_ref, qb_ref, qc_ref, g1_ref, g2_ref, g3_ref, o_ref):
    g1 = g1_ref[...]
    g1b = S_SCALE * jnp.sum(g1 * g1, axis=1)[None, :]
    m0 = jax.lax.dot_general(
        e1s_ref[...], g1, (((1,), (1,)), ((), ())),
        preferred_element_type=jnp.float32)
    w = m0 + qb_ref[...] + g1b
    t = C_THIRD - w * w
    cer3 = jnp.where(t >= T_CUT, jnp.exp2(t), 0.0)

    g2 = g2_ref[...]
    g3 = g3_ref[...]
    r2 = 1.0 / (jnp.sqrt(jnp.sum(g2 * g2, axis=1, keepdims=True)) + 1e-12)
    r3 = 1.0 / (jnp.sqrt(jnp.sum(g3 * g3, axis=1, keepdims=True)) + 1e-12)
    gc = jnp.concatenate([g2 * r2, g3 * r3], axis=1)
    ccos = jax.lax.dot_general(
        qc_ref[...], gc, (((1,), (1,)), ((), ())),
        preferred_element_type=jnp.float32)
    o_ref[...] = cer3 + ccos


@functools.partial(jax.jit, static_argnames=("interpret",))
def kernel(x, W1, W2, W3, G1, G2, G3, interpret=False):
    e1s, qb, qc = pl.pallas_call(
        _embed_kernel,
        out_shape=[
            jax.ShapeDtypeStruct((Q, D_EMB), jnp.float32),
            jax.ShapeDtypeStruct((Q, 1), jnp.float32),
            jax.ShapeDtypeStruct((Q, 2 * D_EMB), jnp.float32),
        ],
        interpret=interpret,
    )(x, W1, W2, W3)

    nblk = pl.cdiv(K_GAL, KBLK)
    gal_spec = pl.BlockSpec((KBLK, D_EMB), lambda k: (k, 0))
    out = pl.pallas_call(
        _main_kernel,
        grid=(nblk,),
        in_specs=[
            pl.BlockSpec((Q, D_EMB), lambda k: (0, 0)),
            pl.BlockSpec((Q, 1), lambda k: (0, 0)),
            pl.BlockSpec((Q, 2 * D_EMB), lambda k: (0, 0)),
            gal_spec, gal_spec, gal_spec,
        ],
        out_specs=pl.BlockSpec((Q, KBLK), lambda k: (0, k)),
        out_shape=jax.ShapeDtypeStruct((Q, K_GAL), jnp.float32),
        compiler_params=pltpu.CompilerParams(
            dimension_semantics=("parallel",)),
        interpret=interpret,
    )(e1s, qb, qc, G1, G2, G3)
    return out
